# Initial kernel scaffold; baseline (speedup 1.0000x reference)
import jax
import jax.numpy as jnp
from jax.experimental import pallas as pl


def kernel(z_e, emb_weight):
    raise NotImplementedError("write your pallas kernel here")


# speedup vs baseline: 1.1180x; 1.1180x over previous
import functools

import jax
import jax.numpy as jnp
from jax.experimental import pallas as pl

NUM_EMBEDDINGS = 1024
EMBEDDING_DIM = 128
BETA = 1.0


def _vq_kernel(z_ref, z2_ref, e2_ref, embT_ref, zq_ref, idx_ref, loss_ref):
    b = pl.program_id(0)
    zt = z_ref[0]
    embT = embT_ref[...]
    P = zt.shape[0]

    m = jnp.dot(zt, embT, preferred_element_type=jnp.float32)
    z2 = z2_ref[0].reshape(P, 1)
    e2 = e2_ref[...]
    dist = (z2 + e2) - 2.0 * m

    mv = jnp.min(dist, axis=1, keepdims=True)
    ci = jax.lax.broadcasted_iota(jnp.int32, dist.shape, 1)
    cand = jnp.where(dist == mv, ci, jnp.int32(NUM_EMBEDDINGS))
    idx = jnp.min(cand, axis=1).astype(jnp.int32)
    idx_row = idx.reshape(1, P)
    idx_ref[0] = idx_row

    code_iota = jax.lax.broadcasted_iota(
        jnp.int32, (NUM_EMBEDDINGS, P), 0)
    oh = (code_iota == idx_row).astype(jnp.float32)
    zq = jax.lax.dot(embT, oh, precision=jax.lax.Precision.HIGHEST)
    zq_ref[0] = zq

    d = zq - zt.T
    part = jnp.sum(d * d, axis=(0, 1), keepdims=True)

    @pl.when(b == 0)
    def _init():
        loss_ref[...] = jnp.zeros_like(loss_ref)

    loss_ref[...] += part


@functools.partial(jax.jit, static_argnames=())
def kernel(z_e, emb_weight):
    B, C, H, W = z_e.shape
    P = H * W
    z_flat = jnp.transpose(z_e, (0, 2, 3, 1)).reshape(-1, C)
    z2 = jnp.sum(z_flat ** 2, axis=1).reshape(B, 1, P)
    e2 = jnp.sum(emb_weight ** 2, axis=1).reshape(1, NUM_EMBEDDINGS)
    z3 = z_flat.reshape(B, P, C)
    embT = emb_weight.T

    zq3, idx3, loss_sum = pl.pallas_call(
        _vq_kernel,
        grid=(B,),
        in_specs=[
            pl.BlockSpec((1, P, C), lambda b: (b, 0, 0)),
            pl.BlockSpec((1, 1, P), lambda b: (b, 0, 0)),
            pl.BlockSpec((1, NUM_EMBEDDINGS), lambda b: (0, 0)),
            pl.BlockSpec((EMBEDDING_DIM, NUM_EMBEDDINGS), lambda b: (0, 0)),
        ],
        out_specs=[
            pl.BlockSpec((1, C, P), lambda b: (b, 0, 0)),
            pl.BlockSpec((1, 1, P), lambda b: (b, 0, 0)),
            pl.BlockSpec((1, 1), lambda b: (0, 0)),
        ],
        out_shape=[
            jax.ShapeDtypeStruct((B, C, P), jnp.float32),
            jax.ShapeDtypeStruct((B, 1, P), jnp.int32),
            jax.ShapeDtypeStruct((1, 1), jnp.float32),
        ],
    )(z3, z2, e2, embT)

    z_q = zq3.reshape(B, C, H, W)
    indices = idx3.reshape(B * P)
    loss = (loss_sum / jnp.float32(z_e.size)).reshape(())
    codebook_loss = loss
    commitment_loss = loss
    vq_loss = codebook_loss + BETA * commitment_loss
    z_q_st = z_q
    return (z_q_st, codebook_loss, commitment_loss, vq_loss, indices)


# speedup vs baseline: 1.4207x; 1.2708x over previous
import functools

import jax
import jax.numpy as jnp
from jax.experimental import pallas as pl

NUM_EMBEDDINGS = 1024
EMBEDDING_DIM = 128
BETA = 1.0


def _vq_kernel(z_ref, z2_ref, e2_ref, emb_ref, hi_ref, lo_ref, ci_ref,
               zq_ref, idx_ref, loss_ref):
    z = z_ref[0]
    emb = emb_ref[...]
    P = z.shape[1]

    mT = jnp.dot(emb, z, preferred_element_type=jnp.float32)
    z2 = z2_ref[0]
    e2 = e2_ref[...]
    dist = (z2 + e2) - 2.0 * mT

    mv = jnp.min(dist, axis=0, keepdims=True)
    cand = jnp.where(dist == mv, ci_ref[...], jnp.float32(NUM_EMBEDDINGS))
    idxf = jnp.min(cand, axis=0, keepdims=True)
    idx_row = idxf.astype(jnp.int32)
    idx_ref[0] = idx_row

    oh = (cand == idxf).astype(jnp.bfloat16)
    zq = (jnp.dot(hi_ref[...], oh, preferred_element_type=jnp.float32)
          + jnp.dot(lo_ref[...], oh, preferred_element_type=jnp.float32))
    zq_ref[0] = zq

    loss_ref[0] = jnp.sum(mv, axis=(0, 1), keepdims=True)


@functools.partial(jax.jit, static_argnames=())
def kernel(z_e, emb_weight):
    B, C, H, W = z_e.shape
    P = H * W
    z_flat = jnp.transpose(z_e, (0, 2, 3, 1)).reshape(-1, C)
    z2 = jnp.sum(z_flat ** 2, axis=1).reshape(B, 1, P)
    e2 = jnp.sum(emb_weight ** 2, axis=1).reshape(NUM_EMBEDDINGS, 1)
    z3 = z_e.reshape(B, C, P)
    embT = emb_weight.T
    embT_hi = embT.astype(jnp.bfloat16)
    embT_lo = (embT - embT_hi.astype(jnp.float32)).astype(jnp.bfloat16)
    cif = jax.lax.broadcasted_iota(
        jnp.float32, (NUM_EMBEDDINGS, P), 0)

    zq3, idx3, loss3 = pl.pallas_call(
        _vq_kernel,
        grid=(B,),
        in_specs=[
            pl.BlockSpec((1, C, P), lambda b: (b, 0, 0)),
            pl.BlockSpec((1, 1, P), lambda b: (b, 0, 0)),
            pl.BlockSpec((NUM_EMBEDDINGS, 1), lambda b: (0, 0)),
            pl.BlockSpec((NUM_EMBEDDINGS, EMBEDDING_DIM), lambda b: (0, 0)),
            pl.BlockSpec((EMBEDDING_DIM, NUM_EMBEDDINGS), lambda b: (0, 0)),
            pl.BlockSpec((EMBEDDING_DIM, NUM_EMBEDDINGS), lambda b: (0, 0)),
            pl.BlockSpec((NUM_EMBEDDINGS, P), lambda b: (0, 0)),
        ],
        out_specs=[
            pl.BlockSpec((1, C, P), lambda b: (b, 0, 0)),
            pl.BlockSpec((1, 1, P), lambda b: (b, 0, 0)),
            pl.BlockSpec((1, 1, 1), lambda b: (b, 0, 0)),
        ],
        out_shape=[
            jax.ShapeDtypeStruct((B, C, P), jnp.float32),
            jax.ShapeDtypeStruct((B, 1, P), jnp.int32),
            jax.ShapeDtypeStruct((B, 1, 1), jnp.float32),
        ],
    )(z3, z2, e2, emb_weight, embT_hi, embT_lo, cif)

    z_q = zq3.reshape(B, C, H, W)
    indices = idx3.reshape(B * P)
    loss = (jnp.sum(loss3) / jnp.float32(z_e.size)).reshape(())
    codebook_loss = loss
    commitment_loss = loss
    vq_loss = codebook_loss + BETA * commitment_loss
    z_q_st = z_q
    return (z_q_st, codebook_loss, commitment_loss, vq_loss, indices)
